# Initial kernel scaffold; baseline (speedup 1.0000x reference)
import jax
import jax.numpy as jnp
from jax.experimental import pallas as pl


def kernel(node_embeddings, target_index, port_index, port_batch, port_weight, pma_seed, pma_wq, pma_wk, pma_wv, pma_wo, ca_wq, ca_wk, ca_wv, ca_wo, ln_g, ln_b, f_w1, f_b1, f_w2, f_b2, h_w1, h_b1, h_w2, h_b2):
    raise NotImplementedError("write your pallas kernel here")


# speedup vs baseline: 44.1550x; 44.1550x over previous
import functools

import jax
import jax.numpy as jnp
from jax import lax
from jax.experimental import pallas as pl
from jax.experimental.pallas import tpu as pltpu
from jax.experimental.pallas import tpu_sc as plsc

D = 128
H = 4
DH = 32
B = 16
N = 65536
P = 32768
NQ = 3
CT = 2048
NB = P // CT
NWORK = 32
ROWS_W = P // NWORK
CH = 128
NEG = -1e30


def _mm(a, b):
    return lax.dot_general(a, b, (((1,), (0,)), ((), ())),
                           precision=lax.Precision.HIGHEST,
                           preferred_element_type=jnp.float32)


def _mmT(a, b):
    return lax.dot_general(a, b, (((0,), (0,)), ((), ())),
                           precision=lax.Precision.HIGHEST,
                           preferred_element_type=jnp.float32)


def _mmRT(a, b):
    return lax.dot_general(a, b, (((1,), (1,)), ((), ())),
                           precision=lax.Precision.HIGHEST,
                           preferred_element_type=jnp.float32)


def _eye(n):
    r = lax.broadcasted_iota(jnp.int32, (n, n), 0)
    c = lax.broadcasted_iota(jnp.int32, (n, n), 1)
    return (r == c).astype(jnp.float32)


def _sc_gather(table, pidx, tidx):
    mesh = plsc.VectorSubcoreMesh(core_axis_name="c", subcore_axis_name="s")

    @functools.partial(
        pl.kernel,
        out_type=[jax.ShapeDtypeStruct((P, D), jnp.float32),
                  jax.ShapeDtypeStruct((B, D), jnp.float32)],
        mesh=mesh,
        scratch_types=[pltpu.VMEM((CH,), jnp.int32),
                       pltpu.VMEM((CH, D), jnp.float32),
                       pltpu.VMEM((B,), jnp.int32),
                       pltpu.VMEM((B, D), jnp.float32),
                       pltpu.SemaphoreType.DMA],
    )
    def k(table_hbm, pidx_hbm, tidx_hbm, out_hbm, tout_hbm,
          idx_v, rows_v, tidx_v, trows_v, sem):
        c = lax.axis_index("c")
        s = lax.axis_index("s")
        wid = s * 2 + c
        base = wid * ROWS_W
        for st in range(ROWS_W // CH):
            off = base + st * CH
            pltpu.sync_copy(pidx_hbm.at[pl.ds(off, CH)], idx_v)
            pltpu.async_copy(table_hbm.at[idx_v], rows_v, sem).wait()
            pltpu.sync_copy(rows_v, out_hbm.at[pl.ds(off, CH)])

        @pl.when(wid == 0)
        def _():
            pltpu.sync_copy(tidx_hbm, tidx_v)
            pltpu.async_copy(table_hbm.at[tidx_v], trows_v, sem).wait()
            pltpu.sync_copy(trows_v, tout_hbm)

    return k(table, pidx, tidx)


def _tc_body(gath, seg3, pw3, tgt, seed, wq, wk, wv, wo,
             cwq, cwk, cwv, cwo, lng, lnb, fw1, fb1, fw2, fb2,
             hw1, hb1, hw2, hb2, out, G, m, l, S):
    i = pl.program_id(0)

    @pl.when(i == 0)
    def _prologue():
        qflat = _mm(seed[...], wq[...])
        e_i = lax.broadcasted_iota(jnp.int32, (D, B * H), 0)
        j_i = lax.broadcasted_iota(jnp.int32, (D, B * H), 1)
        mhead = ((e_i // DH) == (j_i % H)).astype(jnp.float32)
        gl = _mm(wk[...] * qflat, mhead)
        qt = _mm(tgt[...], cwq[...])
        qtT = _mmRT(_eye(D), qt)
        b_i = lax.broadcasted_iota(jnp.int32, (B, B * H), 0)
        j2_i = lax.broadcasted_iota(jnp.int32, (B, B * H), 1)
        rep = ((j2_i // H) == b_i).astype(jnp.float32)
        R = _mm(qtT, rep) * mhead
        gr = _mm(cwk[...], R)
        G[...] = jnp.concatenate([gl, gr], axis=1) * (1.0 / (DH ** 0.5))
        m[...] = jnp.full((1, D), NEG, jnp.float32)
        l[...] = jnp.zeros((1, D), jnp.float32)
        S[...] = jnp.zeros((D, D), jnp.float32)

    T = gath[...]
    segf = seg3[0].astype(jnp.float32)
    logw = jnp.log(pw3[0] + 1e-8)
    ones_r = jnp.ones((1, D), jnp.float32)
    segB = _mmT(segf, ones_r)
    logwB = _mmT(logw, ones_r)
    jj = lax.broadcasted_iota(jnp.int32, (CT, D), 1)
    bcol = ((jj // H) % B).astype(jnp.float32)
    mask = segB == bcol
    E = _mm(T, G[...]) + logwB
    Em = jnp.where(mask, E, NEG)
    colmax = jnp.max(Em, axis=0, keepdims=True)
    m_old = m[...]
    m_new = jnp.maximum(m_old, colmax)
    alpha = jnp.exp(m_old - m_new)
    e = jnp.where(mask, jnp.exp(E - m_new), 0.0)
    l[...] = l[...] * alpha + jnp.sum(e, axis=0, keepdims=True)
    eye = _eye(D)
    S[...] = _mm(eye * alpha, S[...]) + _mmT(e, T)
    m[...] = m_new

    @pl.when(i == NB - 1)
    def _epilogue():
        lv = l[...]
        rinv = 1.0 / (lv + 1e-9)
        Sn = _mm(_eye(D) * rinv, S[...])
        Sn0 = Sn[0:B * H, :]
        Sn1 = Sn[B * H:2 * B * H, :]
        r64 = lax.broadcasted_iota(jnp.int32, (B * H, D), 0)
        d64 = lax.broadcasted_iota(jnp.int32, (B * H, D), 1)
        hmask = ((r64 % H) == (d64 // DH)).astype(jnp.float32)
        bb = lax.broadcasted_iota(jnp.int32, (B, B * H), 0)
        cc = lax.broadcasted_iota(jnp.int32, (B, B * H), 1)
        red = ((cc // H) == bb).astype(jnp.float32)
        ctx = _mm(red, _mm(Sn0, wv[...]) * hmask)
        contexts = _mm(ctx, wo[...])
        fus = _mm(red, _mm(Sn1, cwv[...]) * hmask)
        fusedmm = _mm(fus, cwo[...])
        bb16 = lax.broadcasted_iota(jnp.int32, (B, D), 0)
        jj16 = lax.broadcasted_iota(jnp.int32, (B, D), 1)
        sel = (jj16 == H * bb16).astype(jnp.float32)
        has16 = _mmRT(sel, lv)
        tg = tgt[...]
        fused = jnp.where(has16 > 0.0, fusedmm, tg)
        z = jnp.concatenate([tg, contexts, fused], axis=1)
        mu = jnp.mean(z, axis=1, keepdims=True)
        zc = z - mu
        var = jnp.mean(zc * zc, axis=1, keepdims=True)
        zn = zc / jnp.sqrt(var + 1e-5) * lng[...] + lnb[...]
        h1 = jnp.maximum(_mm(zn, fw1[...]) + fb1[...], 0.0)
        h2 = _mm(h1, fw2[...]) + fb2[...]
        o1 = jnp.maximum(_mm(h2, hw1[...]) + hb1[...], 0.0)
        out[...] = _mm(o1, hw2[...]) + hb2[...]


def _tc_main(gathered, seg3, pw3, targets, seed2, pma_wq, pma_wk, pma_wv,
             pma_wo, ca_wq, ca_wk, ca_wv, ca_wo, lng, lnb, f_w1, fb1, f_w2,
             fb2, h_w1, hb1, h_w2p, hb2p):
    full = lambda shape: pl.BlockSpec(shape, lambda i: (0,) * len(shape))
    return pl.pallas_call(
        _tc_body,
        grid=(NB,),
        in_specs=[
            pl.BlockSpec((CT, D), lambda i: (i, 0)),
            pl.BlockSpec((1, 1, CT), lambda i: (i, 0, 0)),
            pl.BlockSpec((1, 1, CT), lambda i: (i, 0, 0)),
            full((B, D)),
            full((1, D)),
            full((D, D)), full((D, D)), full((D, D)), full((D, D)),
            full((D, D)), full((D, D)), full((D, D)), full((D, D)),
            full((1, 3 * D)), full((1, 3 * D)),
            full((3 * D, D)), full((1, D)),
            full((D, D)), full((1, D)),
            full((D, 2 * D)), full((1, 2 * D)),
            full((2 * D, D)), full((1, D)),
        ],
        out_specs=pl.BlockSpec((B, D), lambda i: (0, 0)),
        out_shape=jax.ShapeDtypeStruct((B, D), jnp.float32),
        scratch_shapes=[
            pltpu.VMEM((D, D), jnp.float32),
            pltpu.VMEM((1, D), jnp.float32),
            pltpu.VMEM((1, D), jnp.float32),
            pltpu.VMEM((D, D), jnp.float32),
        ],
    )(gathered, seg3, pw3, targets, seed2, pma_wq, pma_wk, pma_wv, pma_wo,
      ca_wq, ca_wk, ca_wv, ca_wo, lng, lnb, f_w1, fb1, f_w2, fb2,
      h_w1, hb1, h_w2p, hb2p)


def kernel(node_embeddings, target_index, port_index, port_batch,
           port_weight, pma_seed, pma_wq, pma_wk, pma_wv, pma_wo,
           ca_wq, ca_wk, ca_wv, ca_wo, ln_g, ln_b, f_w1, f_b1, f_w2, f_b2,
           h_w1, h_b1, h_w2, h_b2):
    gathered, targets = _sc_gather(node_embeddings, port_index, target_index)
    seg3 = port_batch.reshape(NB, 1, CT)
    pw3 = port_weight.reshape(NB, 1, CT)
    seed2 = pma_seed.reshape(1, D)
    lng = ln_g.reshape(1, 3 * D)
    lnb = ln_b.reshape(1, 3 * D)
    fb1 = f_b1.reshape(1, D)
    fb2 = f_b2.reshape(1, D)
    hb1 = h_b1.reshape(1, 2 * D)
    h_w2p = jnp.pad(h_w2, ((0, 0), (0, D - NQ)))
    hb2p = jnp.pad(h_b2, (0, D - NQ)).reshape(1, D)
    out128 = _tc_main(gathered, seg3, pw3, targets, seed2, pma_wq, pma_wk,
                      pma_wv, pma_wo, ca_wq, ca_wk, ca_wv, ca_wo, lng, lnb,
                      f_w1, fb1, f_w2, fb2, h_w1, hb1, h_w2p, hb2p)
    return out128[:, :NQ]
